# baseline (speedup 1.0000x reference)
import functools

import jax
import jax.numpy as jnp
from jax import lax
from jax.experimental import pallas as pl
from jax.experimental.pallas import tpu as pltpu
from jax.experimental.pallas import tpu_sc as plsc

N_VAR = 1000
D_MODEL = 128
B = 4096
F = 100

TOT = B * F
NC, NS = 2, 16
NW = NC * NS
PER_W = TOT // NW
CHUNK = 128
N_CHUNKS = PER_W // CHUNK
NBUF = 4
N_GROUPS = N_CHUNKS // NBUF

_mesh = plsc.VectorSubcoreMesh(core_axis_name="c", subcore_axis_name="s")


@functools.partial(
    pl.kernel,
    mesh=_mesh,
    out_type=jax.ShapeDtypeStruct((TOT, D_MODEL), jnp.float32),
    scratch_types=[
        pltpu.VMEM((F, CHUNK), jnp.int32),
        pltpu.VMEM((NBUF, CHUNK, D_MODEL), jnp.float32),
        pltpu.VMEM_SHARED((N_VAR, D_MODEL), jnp.float32),
        pltpu.SemaphoreType.DMA((NBUF,)),
        pltpu.SemaphoreType.DMA((NBUF,)),
    ],
)
def _emb_lookup(table_hbm, idx_hbm, out_hbm, idx_v, rows_v, table_sp, gsem, ssem):
    sid = lax.axis_index("s")
    wid = sid * NC + lax.axis_index("c")
    col0 = wid * CHUNK

    @pl.when(sid == 0)
    def _():
        pltpu.sync_copy(table_hbm, table_sp)

    pltpu.sync_copy(idx_hbm.at[:, pl.ds(col0, CHUNK)], idx_v)
    plsc.subcore_barrier()

    def start_gather(g, b):
        src = table_hbm if b == NBUF - 1 else table_sp
        pltpu.async_copy(src.at[idx_v.at[g]], rows_v.at[b], gsem.at[b])

    def wait_gather(b):
        pltpu.make_async_copy(
            table_sp.at[pl.ds(0, CHUNK)], rows_v.at[b], gsem.at[b]
        ).wait()

    def out_slice(g):
        return out_hbm.at[pl.ds(g * B + col0, CHUNK)]

    def start_store(g, b):
        pltpu.async_copy(rows_v.at[b], out_slice(g), ssem.at[b])

    def wait_store(b):
        pltpu.make_async_copy(
            rows_v.at[b], out_hbm.at[pl.ds(0, CHUNK)], ssem.at[b]
        ).wait()

    for b in range(NBUF):
        start_gather(b, b)

    def group(g0, carry):
        for b in range(NBUF):
            wait_gather(b)
            start_store(g0 * NBUF + b, b)
        for b in range(NBUF):
            wait_store(b)
            start_gather(g0 * NBUF + b + NBUF, b)
        return carry

    lax.fori_loop(0, N_GROUPS - 1, group, 0)

    for b in range(NBUF):
        g = (N_GROUPS - 1) * NBUF + b
        wait_gather(b)
        pltpu.sync_copy(rows_v.at[b], out_slice(g))


def kernel(x, table):
    idx = jnp.transpose(x).astype(jnp.int32)
    out = _emb_lookup(table, idx)
    return jnp.swapaxes(out.reshape(F, B, D_MODEL), 0, 1)


# speedup vs baseline: 1.2858x; 1.2858x over previous
import functools

import jax
import jax.numpy as jnp
from jax import lax
from jax.experimental import pallas as pl
from jax.experimental.pallas import tpu as pltpu
from jax.experimental.pallas import tpu_sc as plsc

N_VAR = 1000
D_MODEL = 128
B = 4096
F = 100

TOT = B * F
NC, NS = 2, 16
NW = NC * NS
PER_W = TOT // NW
CHUNK = 128
N_CHUNKS = PER_W // CHUNK
NBUF = 4
N_GROUPS = N_CHUNKS // NBUF

_mesh = plsc.VectorSubcoreMesh(core_axis_name="c", subcore_axis_name="s")


@functools.partial(
    pl.kernel,
    mesh=_mesh,
    out_type=jax.ShapeDtypeStruct((TOT, D_MODEL), jnp.float32),
    scratch_types=[
        pltpu.VMEM((F, CHUNK), jnp.int32),
        pltpu.VMEM((NBUF, CHUNK, D_MODEL), jnp.float32),
        pltpu.VMEM_SHARED((N_VAR, D_MODEL), jnp.float32),
        pltpu.SemaphoreType.DMA((NBUF,)),
        pltpu.SemaphoreType.DMA((NBUF,)),
    ],
)
def _emb_lookup(table_hbm, idx_hbm, out_hbm, idx_v, rows_v, table_sp, gsem, ssem):
    sid = lax.axis_index("s")
    wid = sid * NC + lax.axis_index("c")
    col0 = wid * CHUNK

    @pl.when(sid == 0)
    def _():
        pltpu.sync_copy(table_hbm, table_sp)

    pltpu.sync_copy(idx_hbm.at[:, pl.ds(col0, CHUNK)], idx_v)
    plsc.subcore_barrier()

    def start_gather(g, b):
        pltpu.async_copy(table_sp.at[idx_v.at[g]], rows_v.at[b], gsem.at[b])

    def wait_gather(b):
        pltpu.make_async_copy(
            table_sp.at[pl.ds(0, CHUNK)], rows_v.at[b], gsem.at[b]
        ).wait()

    def out_slice(g):
        return out_hbm.at[pl.ds(g * B + col0, CHUNK)]

    def start_store(g, b):
        pltpu.async_copy(rows_v.at[b], out_slice(g), ssem.at[b])

    def wait_store(b):
        pltpu.make_async_copy(
            rows_v.at[b], out_hbm.at[pl.ds(0, CHUNK)], ssem.at[b]
        ).wait()

    for b in range(NBUF):
        start_gather(b, b)

    def group(g0, carry):
        for b in range(NBUF):
            wait_gather(b)
            start_store(g0 * NBUF + b, b)
        for b in range(NBUF):
            wait_store(b)
            start_gather(g0 * NBUF + b + NBUF, b)
        return carry

    lax.fori_loop(0, N_GROUPS - 1, group, 0)

    for b in range(NBUF):
        g = (N_GROUPS - 1) * NBUF + b
        wait_gather(b)
        pltpu.sync_copy(rows_v.at[b], out_slice(g))


def kernel(x, table):
    idx = jnp.transpose(x).astype(jnp.int32)
    out = _emb_lookup(table, idx)
    return jnp.swapaxes(out.reshape(F, B, D_MODEL), 0, 1)
